# baseline (speedup 1.0000x reference)
import functools

import jax
import jax.numpy as jnp
from jax import lax
from jax.experimental import pallas as pl
from jax.experimental.pallas import tpu as pltpu
from jax.experimental.pallas import tpu_sc as plsc

VOCAB_ = 1000000
D_ = 64
B_ = 4096 * 200
NC_ = 2
NS_ = 16
NW_ = NC_ * NS_
BPW_ = B_ // NW_
CHUNK_ = 512
NCHUNK_ = BPW_ // CHUNK_


def _emb_body(x_hbm, table_hbm, out_hbm, idx_v, rows_v, sem):
    wid = lax.axis_index("s") * NC_ + lax.axis_index("c")
    base = pl.multiple_of(wid * BPW_, BPW_)
    pltpu.sync_copy(x_hbm.at[pl.ds(base, BPW_)], idx_v)

    def step(i, carry):
        off = pl.multiple_of(i * CHUNK_, CHUNK_)
        pltpu.async_copy(
            table_hbm.at[idx_v.at[pl.ds(off, CHUNK_)]], rows_v, sem
        ).wait()
        pltpu.sync_copy(rows_v, out_hbm.at[pl.ds(base + off, CHUNK_)])
        return carry

    lax.fori_loop(0, NCHUNK_, step, 0)


def _emb(x_flat, table):
    mesh = plsc.VectorSubcoreMesh(core_axis_name="c", subcore_axis_name="s")
    k = pl.kernel(
        _emb_body,
        out_type=jax.ShapeDtypeStruct((B_, D_), jnp.float32),
        mesh=mesh,
        scratch_types=[
            pltpu.VMEM((BPW_,), jnp.int32),
            pltpu.VMEM((CHUNK_, D_), jnp.float32),
            pltpu.SemaphoreType.DMA,
        ],
        compiler_params=pltpu.CompilerParams(use_tc_tiling_on_sc=False),
    )
    return k(x_flat, table)


def kernel(x, table):
    x_flat = x.reshape(-1).astype(jnp.int32)
    out = _emb(x_flat, table)
    return out.reshape(x.shape + (D_,))


# speedup vs baseline: 1.0217x; 1.0217x over previous
import functools

import jax
import jax.numpy as jnp
from jax import lax
from jax.experimental import pallas as pl
from jax.experimental.pallas import tpu as pltpu
from jax.experimental.pallas import tpu_sc as plsc

VOCAB_ = 1000000
D_ = 64
B_ = 4096 * 200
NC_ = 2
NS_ = 16
NW_ = NC_ * NS_
BPW_ = B_ // NW_
CHUNK_ = 256
NBUF_ = 4
NCHUNK_ = BPW_ // CHUNK_
assert NCHUNK_ % NBUF_ == 0


def _emb_body(x_hbm, table_hbm, out_hbm, idx_v, rows_v, gsems, ssems):
    wid = lax.axis_index("s") * NC_ + lax.axis_index("c")
    base = pl.multiple_of(wid * BPW_, BPW_)
    pltpu.sync_copy(x_hbm.at[pl.ds(base, BPW_)], idx_v)

    def gather_start(b, j):
        off = pl.multiple_of(j * CHUNK_, CHUNK_)
        pltpu.async_copy(
            table_hbm.at[idx_v.at[pl.ds(off, CHUNK_)]], rows_v.at[b], gsems[b]
        )

    def gather_wait(b):
        pltpu.make_async_copy(
            table_hbm.at[idx_v.at[pl.ds(0, CHUNK_)]], rows_v.at[b], gsems[b]
        ).wait()

    def store_start(b, j):
        off = pl.multiple_of(j * CHUNK_, CHUNK_)
        pltpu.async_copy(
            rows_v.at[b], out_hbm.at[pl.ds(base + off, CHUNK_)], ssems[b]
        )

    def store_wait(b):
        pltpu.make_async_copy(
            rows_v.at[b], out_hbm.at[pl.ds(base, CHUNK_)], ssems[b]
        ).wait()

    for b in range(NBUF_):
        gather_start(b, b)

    @pl.loop(0, NCHUNK_ - NBUF_, step=NBUF_)
    def _round(i):
        for b in range(NBUF_):
            gather_wait(b)
            store_start(b, i + b)
        for b in range(NBUF_):
            store_wait(b)
            gather_start(b, i + b + NBUF_)

    for b in range(NBUF_):
        gather_wait(b)
        store_start(b, NCHUNK_ - NBUF_ + b)
    for b in range(NBUF_):
        store_wait(b)


def _emb(x_flat, table):
    mesh = plsc.VectorSubcoreMesh(core_axis_name="c", subcore_axis_name="s")
    k = pl.kernel(
        _emb_body,
        out_type=jax.ShapeDtypeStruct((B_, D_), jnp.float32),
        mesh=mesh,
        scratch_types=[
            pltpu.VMEM((BPW_,), jnp.int32),
            pltpu.VMEM((NBUF_, CHUNK_, D_), jnp.float32),
            [pltpu.SemaphoreType.DMA] * NBUF_,
            [pltpu.SemaphoreType.DMA] * NBUF_,
        ],
        compiler_params=pltpu.CompilerParams(use_tc_tiling_on_sc=False),
    )
    return k(x_flat, table)


def kernel(x, table):
    x_flat = x.reshape(-1).astype(jnp.int32)
    out = _emb(x_flat, table)
    return out.reshape(x.shape + (D_,))


# speedup vs baseline: 1.0904x; 1.0672x over previous
import jax
import jax.numpy as jnp
from jax import lax
from jax.experimental import pallas as pl
from jax.experimental.pallas import tpu as pltpu
from jax.experimental.pallas import tpu_sc as plsc

VOCAB_ = 1000000
D_ = 64
DP_ = 128
B_ = 4096 * 200
NC_ = 2
NS_ = 16
NW_ = NC_ * NS_
BPW_ = B_ // NW_
CHUNK_ = 128
NCHUNK_ = BPW_ // CHUNK_


def _emb_body(x_hbm, table_hbm, out_hbm, idx_v, rows_v, gsem):
    wid = lax.axis_index("s") * NC_ + lax.axis_index("c")
    cbase = wid * NCHUNK_
    base = pl.multiple_of(wid * BPW_, BPW_)
    pltpu.sync_copy(x_hbm.at[pl.ds(cbase, NCHUNK_)], idx_v)

    @pl.loop(0, NCHUNK_)
    def _chunk(j):
        pltpu.async_copy(table_hbm.at[idx_v.at[j]], rows_v, gsem).wait()
        pltpu.sync_copy(rows_v, out_hbm.at[pl.ds(base + j * CHUNK_, CHUNK_)])


def _emb(x_2d, table_p):
    mesh = plsc.VectorSubcoreMesh(core_axis_name="c", subcore_axis_name="s")
    k = pl.kernel(
        _emb_body,
        out_type=jax.ShapeDtypeStruct((B_, DP_), jnp.float32),
        mesh=mesh,
        scratch_types=[
            pltpu.VMEM((NCHUNK_, CHUNK_), jnp.int32),
            pltpu.VMEM((CHUNK_, DP_), jnp.float32),
            pltpu.SemaphoreType.DMA,
        ],
    )
    return k(x_2d, table_p)


def kernel(x, table):
    x_2d = x.reshape(B_ // CHUNK_, CHUNK_).astype(jnp.int32)
    table_p = jnp.pad(table, ((0, 0), (0, DP_ - D_)))
    out = _emb(x_2d, table_p)
    return out[:, :D_].reshape(x.shape + (D_,))


# speedup vs baseline: 1.2490x; 1.1455x over previous
import jax
import jax.numpy as jnp
from jax import lax
from jax.experimental import pallas as pl
from jax.experimental.pallas import tpu as pltpu
from jax.experimental.pallas import tpu_sc as plsc

VOCAB_ = 1000000
D_ = 64
DP_ = 128
B_ = 4096 * 200
NC_ = 2
NS_ = 16
NW_ = NC_ * NS_
BPW_ = B_ // NW_
CHUNK_ = 128
NCHUNK_ = BPW_ // CHUNK_


NBUF_ = 4
assert NCHUNK_ % NBUF_ == 0


def _emb_body(x_hbm, table_hbm, out_hbm, idx_v, rows_v, gsems, ssems):
    wid = lax.axis_index("s") * NC_ + lax.axis_index("c")
    cbase = wid * NCHUNK_
    base = pl.multiple_of(wid * BPW_, BPW_)
    pltpu.sync_copy(x_hbm.at[pl.ds(cbase, NCHUNK_)], idx_v)

    def gather_start(b, j):
        pltpu.async_copy(table_hbm.at[idx_v.at[j]], rows_v.at[b], gsems[b])

    def gather_wait(b):
        pltpu.make_async_copy(
            table_hbm.at[idx_v.at[0]], rows_v.at[b], gsems[b]
        ).wait()

    def store_start(b, j):
        pltpu.async_copy(
            rows_v.at[b], out_hbm.at[pl.ds(base + j * CHUNK_, CHUNK_)], ssems[b]
        )

    def store_wait(b):
        pltpu.make_async_copy(
            rows_v.at[b], out_hbm.at[pl.ds(base, CHUNK_)], ssems[b]
        ).wait()

    for b in range(NBUF_):
        gather_start(b, b)

    @pl.loop(0, NCHUNK_ - NBUF_, step=NBUF_)
    def _round(i):
        for b in range(NBUF_):
            gather_wait(b)
            store_start(b, i + b)
        for b in range(NBUF_):
            store_wait(b)
            gather_start(b, i + b + NBUF_)

    for b in range(NBUF_):
        gather_wait(b)
        store_start(b, NCHUNK_ - NBUF_ + b)
    for b in range(NBUF_):
        store_wait(b)


def _emb(x_2d, table_p):
    mesh = plsc.VectorSubcoreMesh(core_axis_name="c", subcore_axis_name="s")
    k = pl.kernel(
        _emb_body,
        out_type=jax.ShapeDtypeStruct((B_, DP_), jnp.float32),
        mesh=mesh,
        scratch_types=[
            pltpu.VMEM((NCHUNK_, CHUNK_), jnp.int32),
            pltpu.VMEM((NBUF_, CHUNK_, DP_), jnp.float32),
            [pltpu.SemaphoreType.DMA] * NBUF_,
            [pltpu.SemaphoreType.DMA] * NBUF_,
        ],
    )
    return k(x_2d, table_p)


def kernel(x, table):
    x_2d = x.reshape(B_ // CHUNK_, CHUNK_).astype(jnp.int32)
    table_p = jnp.pad(table, ((0, 0), (0, DP_ - D_)))
    out = _emb(x_2d, table_p)
    return out[:, :D_].reshape(x.shape + (D_,))
